# Initial kernel scaffold; baseline (speedup 1.0000x reference)
import jax
import jax.numpy as jnp
from jax.experimental import pallas as pl


def kernel(x, edge_index, batch, Wl1, bl1, Wr1, br1, Wl2, bl2, Wr2, br2, Wp1, bp1, Wp2, bp2):
    raise NotImplementedError("write your pallas kernel here")


# speedup vs baseline: 5.5126x; 5.5126x over previous
import functools

import jax
import jax.numpy as jnp
from jax import lax
from jax.experimental import pallas as pl
from jax.experimental.pallas import tpu as pltpu
from jax.experimental.pallas import tpu_sc as plsc

_N = 10000
_E = 320000
_D = 128
_DOUT = 64

_NC = 2
_NS = 16
_NW = _NC * _NS
_EW = _E // _NW
_CHUNK = 80
_CPS = 25
_NSEG = _EW // (_CPS * _CHUNK)
_NP = 10240
_RPT = _NP // _NS
_DEGW = 16

def _segsum_body(with_deg, *refs):
    if with_deg:
        (x_hbm, src_hbm, dst_hbm, out_hbm, deg_hbm,
         src_v, dst_v, row_v, deg_v, acc_sh, sem) = refs
    else:
        (x_hbm, src_hbm, dst_hbm, out_hbm,
         src_v, dst_v, row_v, acc_sh, sem) = refs

    cid = lax.axis_index("c")
    sid = lax.axis_index("s")
    wid = sid * _NC + cid
    base = wid * _EW

    def zb_body(i, _):
        r = i // 8
        c = (i % 8) * 16
        row_v[r, pl.ds(c, 16)] = jnp.zeros((16,), jnp.float32)
        return 0
    lax.fori_loop(0, _CHUNK * 8, zb_body, 0)

    if with_deg:
        def zd_body(i, _):
            deg_v[pl.ds(i * 16, 16)] = jnp.zeros((16,), jnp.float32)
            return 0
        lax.fori_loop(0, _NP // 16, zd_body, 0)

    for k in range(_RPT // _CHUNK):
        pltpu.sync_copy(row_v, acc_sh.at[pl.ds(sid * _RPT + k * _CHUNK, _CHUNK)])
    plsc.subcore_barrier()

    def seg_body(s, _):
        seg_base = base + s * (_CPS * _CHUNK)

        def st_body(j, _):
            off = seg_base + j * _CHUNK
            pltpu.sync_copy(src_hbm.at[pl.ds(off, _CHUNK)], src_v.at[j])
            pltpu.sync_copy(dst_hbm.at[pl.ds(off, _CHUNK)], dst_v.at[j])
            return 0
        lax.fori_loop(0, _CPS, st_body, 0)

        ones16 = jnp.ones((16,), jnp.float32)

        def body(j, _):
            pltpu.async_copy(x_hbm.at[src_v.at[j]], row_v, sem).wait()
            pltpu.sync_copy(row_v, acc_sh.at[dst_v.at[j]], add=True)
            if with_deg:
                for g in range(_CHUNK // 16):
                    idx = dst_v[j, pl.ds(g * 16, 16)]
                    plsc.addupdate_scatter(deg_v, [idx], ones16)
            return 0
        lax.fori_loop(0, _CPS, body, 0)
        return 0
    lax.fori_loop(0, _NSEG, seg_body, 0)

    plsc.subcore_barrier()

    pltpu.sync_copy(acc_sh.at[pl.ds(sid * _RPT, _RPT)],
                    out_hbm.at[cid, pl.ds(sid * _RPT, _RPT)])
    if with_deg:
        pltpu.sync_copy(deg_v, deg_hbm.at[wid])


def _segsum(x, src, dst, with_deg):
    mesh = plsc.VectorSubcoreMesh(core_axis_name="c", subcore_axis_name="s")
    out_type = [jax.ShapeDtypeStruct((_NC, _NP, _D), jnp.float32)]
    scratch = [
        pltpu.VMEM((_CPS, _CHUNK), jnp.int32),
        pltpu.VMEM((_CPS, _CHUNK), jnp.int32),
        pltpu.VMEM((_CHUNK, _D), jnp.float32),
    ]
    if with_deg:
        out_type.append(jax.ShapeDtypeStruct((_NW, _NP), jnp.float32))
        scratch.append(pltpu.VMEM((_NP,), jnp.float32))
    scratch.append(pltpu.VMEM_SHARED((_NP, _D), jnp.float32))
    scratch.append(pltpu.SemaphoreType.DMA)

    k = pl.kernel(
        functools.partial(_segsum_body, with_deg),
        out_type=tuple(out_type),
        mesh=mesh,
        scratch_types=tuple(scratch),
        compiler_params=pltpu.CompilerParams(needs_layout_passes=False),
    )
    return k(x, src, dst)


def _layer1_body(x_ref, s_ref, d_ref, wl_ref, wr_ref, b_ref, o_ref):
    s = s_ref[0] + s_ref[1]
    deg = jnp.sum(d_ref[...], axis=1, keepdims=True)
    agg = s / jnp.maximum(deg, 1.0)
    out = (jnp.dot(x_ref[...], wl_ref[...], preferred_element_type=jnp.float32)
           + jnp.dot(agg, wr_ref[...], preferred_element_type=jnp.float32)
           + b_ref[...])
    norm = jnp.sqrt(jnp.sum(out * out, axis=1, keepdims=True))
    out = out / jnp.maximum(norm, 1e-12)
    o_ref[...] = jnp.maximum(out, 0.0)


def _layer2_body(h_ref, s_ref, d_ref, wl_ref, wr_ref, b_ref,
                 wp1_ref, bp1_ref, wp2_ref, bp2_ref, o_ref):
    s = s_ref[0] + s_ref[1]
    deg = jnp.sum(d_ref[...], axis=1, keepdims=True)
    agg = s / jnp.maximum(deg, 1.0)
    out = (jnp.dot(h_ref[...], wl_ref[...], preferred_element_type=jnp.float32)
           + jnp.dot(agg, wr_ref[...], preferred_element_type=jnp.float32)
           + b_ref[...])
    norm = jnp.sqrt(jnp.sum(out * out, axis=1, keepdims=True))
    out = out / jnp.maximum(norm, 1e-12)
    out = jnp.maximum(out, 0.0)
    m = jnp.dot(out, wp1_ref[...], preferred_element_type=jnp.float32) + bp1_ref[...]
    z = jnp.dot(m, wp2_ref[...], preferred_element_type=jnp.float32) + bp2_ref[...]
    zm = z - jnp.max(z, axis=1, keepdims=True)
    o_ref[...] = zm - jnp.log(jnp.sum(jnp.exp(zm), axis=1, keepdims=True))


_BLK = 1000
_GRID = _N // _BLK


def _layer1(x, s1, degp, Wl, Wr, b):
    return pl.pallas_call(
        _layer1_body,
        grid=(_GRID,),
        in_specs=[
            pl.BlockSpec((_BLK, _D), lambda i: (i, 0)),
            pl.BlockSpec((_NC, _BLK, _D), lambda i: (0, i, 0)),
            pl.BlockSpec((_BLK, _NW), lambda i: (i, 0)),
            pl.BlockSpec((_D, _D), lambda i: (0, 0)),
            pl.BlockSpec((_D, _D), lambda i: (0, 0)),
            pl.BlockSpec((1, _D), lambda i: (0, 0)),
        ],
        out_specs=pl.BlockSpec((_BLK, _D), lambda i: (i, 0)),
        out_shape=jax.ShapeDtypeStruct((_N, _D), jnp.float32),
    )(x, s1, degp, Wl, Wr, b)


def _layer2(h, s2, degp, Wl, Wr, b, Wp1, bp1, Wp2, bp2):
    return pl.pallas_call(
        _layer2_body,
        grid=(_GRID,),
        in_specs=[
            pl.BlockSpec((_BLK, _D), lambda i: (i, 0)),
            pl.BlockSpec((_NC, _BLK, _D), lambda i: (0, i, 0)),
            pl.BlockSpec((_BLK, _NW), lambda i: (i, 0)),
            pl.BlockSpec((_D, _D), lambda i: (0, 0)),
            pl.BlockSpec((_D, _D), lambda i: (0, 0)),
            pl.BlockSpec((1, _D), lambda i: (0, 0)),
            pl.BlockSpec((_D, _D), lambda i: (0, 0)),
            pl.BlockSpec((1, _D), lambda i: (0, 0)),
            pl.BlockSpec((_D, _DOUT), lambda i: (0, 0)),
            pl.BlockSpec((1, _DOUT), lambda i: (0, 0)),
        ],
        out_specs=pl.BlockSpec((_BLK, _DOUT), lambda i: (i, 0)),
        out_shape=jax.ShapeDtypeStruct((_N, _DOUT), jnp.float32),
    )(h, s2, degp, Wl, Wr, b, Wp1, bp1, Wp2, bp2)


def kernel(x, edge_index, batch, Wl1, bl1, Wr1, br1, Wl2, bl2, Wr2, br2,
           Wp1, bp1, Wp2, bp2):
    src = edge_index[0]
    dst = edge_index[1]
    s1, degp = _segsum(x, src, dst, True)
    degp = degp.T
    h = _layer1(x, s1, degp, Wl1, Wr1, (bl1 + br1).reshape(1, _D))
    (s2,) = _segsum(h, src, dst, False)
    return _layer2(h, s2, degp, Wl2, Wr2, (bl2 + br2).reshape(1, _D),
                   Wp1, bp1.reshape(1, _D), Wp2, bp2.reshape(1, _DOUT))


# speedup vs baseline: 9.5381x; 1.7303x over previous
import functools

import jax
import jax.numpy as jnp
from jax import lax
from jax.experimental import pallas as pl
from jax.experimental.pallas import tpu as pltpu
from jax.experimental.pallas import tpu_sc as plsc

_N = 10000
_E = 320000
_D = 128
_DOUT = 64

_NC = 2
_NS = 16
_NW = _NC * _NS
_EW = _E // _NW
_CHUNK = 80
_CPS = 25
_NSEG = _EW // (_CPS * _CHUNK)
_NP = 10240
_RPT = _NP // _NS
_DEGW = 16

def _segsum_body(with_deg, *refs):
    if with_deg:
        (x_hbm, src_hbm, dst_hbm, out_hbm, deg_hbm,
         srcf_v, dstf_v, dst_v, row2_v, deg_v, acc_sh, gsem0, gsem1) = refs
    else:
        (x_hbm, src_hbm, dst_hbm, out_hbm,
         srcf_v, dstf_v, dst_v, row2_v, acc_sh, gsem0, gsem1) = refs
    row_v = row2_v.at[0]

    cid = lax.axis_index("c")
    sid = lax.axis_index("s")
    wid = sid * _NC + cid
    base = wid * _EW

    def zb_body(i, _):
        r = i // 8
        c = (i % 8) * 16
        row_v[r, pl.ds(c, 16)] = jnp.zeros((16,), jnp.float32)
        return 0
    lax.fori_loop(0, _CHUNK * 8, zb_body, 0)

    if with_deg:
        def zd_body(i, _):
            deg_v[pl.ds(i * 16, 16)] = jnp.zeros((16,), jnp.float32)
            return 0
        lax.fori_loop(0, _NP // 16, zd_body, 0)

    for k in range(_RPT // _CHUNK):
        pltpu.sync_copy(row_v, acc_sh.at[pl.ds(sid * _RPT + k * _CHUNK, _CHUNK)])
    plsc.subcore_barrier()

    _SEGE = _CPS * _CHUNK
    ones16 = jnp.ones((16,), jnp.float32)
    gsems = (gsem0, gsem1)

    def seg_body(s, _):
        seg_base = base + s * _SEGE
        pltpu.sync_copy(src_hbm.at[pl.ds(seg_base, _SEGE)], srcf_v)
        pltpu.sync_copy(dst_hbm.at[pl.ds(seg_base, _SEGE)], dstf_v)

        def mv_body(i, _):
            r = i // (_CHUNK // 16)
            c = (i % (_CHUNK // 16)) * 16
            dst_v[r, pl.ds(c, 16)] = dstf_v[pl.ds(i * 16, 16)]
            return 0
        lax.fori_loop(0, _SEGE // 16, mv_body, 0)

        def gissue(j, b):
            pltpu.async_copy(
                x_hbm.at[srcf_v.at[pl.ds(j * _CHUNK, _CHUNK)]],
                row2_v.at[b], gsems[b])

        def gwait(j, b):
            pltpu.make_async_copy(
                x_hbm.at[srcf_v.at[pl.ds(j * _CHUNK, _CHUNK)]],
                row2_v.at[b], gsems[b]).wait()

        def consume(j, b):
            pltpu.sync_copy(row2_v.at[b], acc_sh.at[dst_v.at[j]], add=True)
            if with_deg:
                for g in range(_CHUNK // 16):
                    idx = dst_v[j, pl.ds(g * 16, 16)]
                    plsc.addupdate_scatter(deg_v, [idx], ones16)

        gissue(0, 0)

        def pipe_body(jj, _):
            j0 = jj * 2
            gwait(j0, 0)
            gissue(j0 + 1, 1)
            consume(j0, 0)
            gwait(j0 + 1, 1)
            gissue(j0 + 2, 0)
            consume(j0 + 1, 1)
            return 0
        lax.fori_loop(0, (_CPS - 1) // 2, pipe_body, 0)

        gwait(_CPS - 1, 0)
        consume(_CPS - 1, 0)
        return 0
    lax.fori_loop(0, _NSEG, seg_body, 0)

    plsc.subcore_barrier()

    pltpu.sync_copy(acc_sh.at[pl.ds(sid * _RPT, _RPT)],
                    out_hbm.at[cid, pl.ds(sid * _RPT, _RPT)])
    if with_deg:
        pltpu.sync_copy(deg_v, deg_hbm.at[wid])


def _segsum(x, src, dst, with_deg):
    mesh = plsc.VectorSubcoreMesh(core_axis_name="c", subcore_axis_name="s")
    out_type = [jax.ShapeDtypeStruct((_NC, _NP, _D), jnp.float32)]
    scratch = [
        pltpu.VMEM((_CPS * _CHUNK,), jnp.int32),
        pltpu.VMEM((_CPS * _CHUNK,), jnp.int32),
        pltpu.VMEM((_CPS, _CHUNK), jnp.int32),
        pltpu.VMEM((2, _CHUNK, _D), jnp.float32),
    ]
    if with_deg:
        out_type.append(jax.ShapeDtypeStruct((_NW, _NP), jnp.float32))
        scratch.append(pltpu.VMEM((_NP,), jnp.float32))
    scratch.append(pltpu.VMEM_SHARED((_NP, _D), jnp.float32))
    scratch.append(pltpu.SemaphoreType.DMA)
    scratch.append(pltpu.SemaphoreType.DMA)

    k = pl.kernel(
        functools.partial(_segsum_body, with_deg),
        out_type=tuple(out_type),
        mesh=mesh,
        scratch_types=tuple(scratch),
        compiler_params=pltpu.CompilerParams(needs_layout_passes=False),
    )
    return k(x, src, dst)


def _layer1_body(x_ref, s_ref, d_ref, wl_ref, wr_ref, b_ref, o_ref):
    s = s_ref[0] + s_ref[1]
    deg = jnp.sum(d_ref[...], axis=1, keepdims=True)
    agg = s / jnp.maximum(deg, 1.0)
    out = (jnp.dot(x_ref[...], wl_ref[...], preferred_element_type=jnp.float32)
           + jnp.dot(agg, wr_ref[...], preferred_element_type=jnp.float32)
           + b_ref[...])
    norm = jnp.sqrt(jnp.sum(out * out, axis=1, keepdims=True))
    out = out / jnp.maximum(norm, 1e-12)
    o_ref[...] = jnp.maximum(out, 0.0)


def _layer2_body(h_ref, s_ref, d_ref, wl_ref, wr_ref, b_ref,
                 wp1_ref, bp1_ref, wp2_ref, bp2_ref, o_ref):
    s = s_ref[0] + s_ref[1]
    deg = jnp.sum(d_ref[...], axis=1, keepdims=True)
    agg = s / jnp.maximum(deg, 1.0)
    out = (jnp.dot(h_ref[...], wl_ref[...], preferred_element_type=jnp.float32)
           + jnp.dot(agg, wr_ref[...], preferred_element_type=jnp.float32)
           + b_ref[...])
    norm = jnp.sqrt(jnp.sum(out * out, axis=1, keepdims=True))
    out = out / jnp.maximum(norm, 1e-12)
    out = jnp.maximum(out, 0.0)
    m = jnp.dot(out, wp1_ref[...], preferred_element_type=jnp.float32) + bp1_ref[...]
    z = jnp.dot(m, wp2_ref[...], preferred_element_type=jnp.float32) + bp2_ref[...]
    zm = z - jnp.max(z, axis=1, keepdims=True)
    o_ref[...] = zm - jnp.log(jnp.sum(jnp.exp(zm), axis=1, keepdims=True))


_BLK = 1000
_GRID = _N // _BLK


def _layer1(x, s1, degp, Wl, Wr, b):
    return pl.pallas_call(
        _layer1_body,
        grid=(_GRID,),
        in_specs=[
            pl.BlockSpec((_BLK, _D), lambda i: (i, 0)),
            pl.BlockSpec((_NC, _BLK, _D), lambda i: (0, i, 0)),
            pl.BlockSpec((_BLK, _NW), lambda i: (i, 0)),
            pl.BlockSpec((_D, _D), lambda i: (0, 0)),
            pl.BlockSpec((_D, _D), lambda i: (0, 0)),
            pl.BlockSpec((1, _D), lambda i: (0, 0)),
        ],
        out_specs=pl.BlockSpec((_BLK, _D), lambda i: (i, 0)),
        out_shape=jax.ShapeDtypeStruct((_N, _D), jnp.float32),
    )(x, s1, degp, Wl, Wr, b)


def _layer2(h, s2, degp, Wl, Wr, b, Wp1, bp1, Wp2, bp2):
    return pl.pallas_call(
        _layer2_body,
        grid=(_GRID,),
        in_specs=[
            pl.BlockSpec((_BLK, _D), lambda i: (i, 0)),
            pl.BlockSpec((_NC, _BLK, _D), lambda i: (0, i, 0)),
            pl.BlockSpec((_BLK, _NW), lambda i: (i, 0)),
            pl.BlockSpec((_D, _D), lambda i: (0, 0)),
            pl.BlockSpec((_D, _D), lambda i: (0, 0)),
            pl.BlockSpec((1, _D), lambda i: (0, 0)),
            pl.BlockSpec((_D, _D), lambda i: (0, 0)),
            pl.BlockSpec((1, _D), lambda i: (0, 0)),
            pl.BlockSpec((_D, _DOUT), lambda i: (0, 0)),
            pl.BlockSpec((1, _DOUT), lambda i: (0, 0)),
        ],
        out_specs=pl.BlockSpec((_BLK, _DOUT), lambda i: (i, 0)),
        out_shape=jax.ShapeDtypeStruct((_N, _DOUT), jnp.float32),
    )(h, s2, degp, Wl, Wr, b, Wp1, bp1, Wp2, bp2)


def kernel(x, edge_index, batch, Wl1, bl1, Wr1, br1, Wl2, bl2, Wr2, br2,
           Wp1, bp1, Wp2, bp2):
    src = edge_index[0]
    dst = edge_index[1]
    s1, degp = _segsum(x, src, dst, True)
    degp = degp.T
    h = _layer1(x, s1, degp, Wl1, Wr1, (bl1 + br1).reshape(1, _D))
    (s2,) = _segsum(h, src, dst, False)
    return _layer2(h, s2, degp, Wl2, Wr2, (bl2 + br2).reshape(1, _D),
                   Wp1, bp1.reshape(1, _D), Wp2, bp2.reshape(1, _DOUT))


# speedup vs baseline: 9.5688x; 1.0032x over previous
import functools

import jax
import jax.numpy as jnp
from jax import lax
from jax.experimental import pallas as pl
from jax.experimental.pallas import tpu as pltpu
from jax.experimental.pallas import tpu_sc as plsc

_N = 10000
_E = 320000
_D = 128
_DOUT = 64

_NC = 2
_NS = 16
_NW = _NC * _NS
_EW = _E // _NW
_CHUNK = 80
_CPS = 25
_NSEG = _EW // (_CPS * _CHUNK)
_NP = 10240
_RPT = _NP // _NS
_DEGW = 16

def _segsum_body(with_deg, *refs):
    if with_deg:
        (x_hbm, src_hbm, dst_hbm, out_hbm, deg_hbm,
         srcf_v, dstf_v, dst_v, row2_v, deg_v, acc_sh,
         gsem0, gsem1, ssem0, ssem1) = refs
    else:
        (x_hbm, src_hbm, dst_hbm, out_hbm,
         srcf_v, dstf_v, dst_v, row2_v, acc_sh,
         gsem0, gsem1, ssem0, ssem1) = refs
    row_v = row2_v.at[0]

    cid = lax.axis_index("c")
    sid = lax.axis_index("s")
    wid = sid * _NC + cid
    base = wid * _EW

    def zb_body(i, _):
        r = i // 8
        c = (i % 8) * 16
        row_v[r, pl.ds(c, 16)] = jnp.zeros((16,), jnp.float32)
        return 0
    lax.fori_loop(0, _CHUNK * 8, zb_body, 0)

    if with_deg:
        def zd_body(i, _):
            deg_v[pl.ds(i * 16, 16)] = jnp.zeros((16,), jnp.float32)
            return 0
        lax.fori_loop(0, _NP // 16, zd_body, 0)

    for k in range(_RPT // _CHUNK):
        pltpu.sync_copy(row_v, acc_sh.at[pl.ds(sid * _RPT + k * _CHUNK, _CHUNK)])
    plsc.subcore_barrier()

    _SEGE = _CPS * _CHUNK
    ones16 = jnp.ones((16,), jnp.float32)
    gsems = (gsem0, gsem1)
    ssems = (ssem0, ssem1)

    def seg_body(s, _):
        seg_base = base + s * _SEGE
        pltpu.sync_copy(src_hbm.at[pl.ds(seg_base, _SEGE)], srcf_v)
        pltpu.sync_copy(dst_hbm.at[pl.ds(seg_base, _SEGE)], dstf_v)

        def mv_body(i, _):
            r = i // (_CHUNK // 16)
            c = (i % (_CHUNK // 16)) * 16
            dst_v[r, pl.ds(c, 16)] = dstf_v[pl.ds(i * 16, 16)]
            return 0
        lax.fori_loop(0, _SEGE // 16, mv_body, 0)

        def gissue(j, b):
            jc = jnp.minimum(j, _CPS - 1)
            pltpu.async_copy(
                x_hbm.at[srcf_v.at[pl.ds(jc * _CHUNK, _CHUNK)]],
                row2_v.at[b], gsems[b])

        def gwait(j, b):
            jc = jnp.minimum(j, _CPS - 1)
            pltpu.make_async_copy(
                x_hbm.at[srcf_v.at[pl.ds(jc * _CHUNK, _CHUNK)]],
                row2_v.at[b], gsems[b]).wait()

        def sissue(j, b):
            pltpu.async_copy(row2_v.at[b], acc_sh.at[dst_v.at[j]],
                             ssems[b], add=True)
            if with_deg:
                for g in range(_CHUNK // 16):
                    idx = dst_v[j, pl.ds(g * 16, 16)]
                    plsc.addupdate_scatter(deg_v, [idx], ones16)

        def swait(j, b):
            pltpu.make_async_copy(row2_v.at[b], acc_sh.at[dst_v.at[j]],
                                  ssems[b]).wait()

        gissue(0, 0)
        gissue(1, 1)

        def pipe_body(jj, _):
            j0 = jj * 2
            gwait(j0, 0)
            sissue(j0, 0)
            gwait(j0 + 1, 1)
            sissue(j0 + 1, 1)
            swait(j0, 0)
            gissue(j0 + 2, 0)
            swait(j0 + 1, 1)
            gissue(j0 + 3, 1)
            return 0
        lax.fori_loop(0, (_CPS - 1) // 2, pipe_body, 0)

        gwait(_CPS - 1, 0)
        sissue(_CPS - 1, 0)
        gwait(_CPS - 1, 1)
        swait(_CPS - 1, 0)
        return 0
    lax.fori_loop(0, _NSEG, seg_body, 0)

    plsc.subcore_barrier()

    pltpu.sync_copy(acc_sh.at[pl.ds(sid * _RPT, _RPT)],
                    out_hbm.at[cid, pl.ds(sid * _RPT, _RPT)])
    if with_deg:
        pltpu.sync_copy(deg_v, deg_hbm.at[wid])


def _segsum(x, src, dst, with_deg):
    mesh = plsc.VectorSubcoreMesh(core_axis_name="c", subcore_axis_name="s")
    out_type = [jax.ShapeDtypeStruct((_NC, _NP, _D), jnp.float32)]
    scratch = [
        pltpu.VMEM((_CPS * _CHUNK,), jnp.int32),
        pltpu.VMEM((_CPS * _CHUNK,), jnp.int32),
        pltpu.VMEM((_CPS, _CHUNK), jnp.int32),
        pltpu.VMEM((2, _CHUNK, _D), jnp.float32),
    ]
    if with_deg:
        out_type.append(jax.ShapeDtypeStruct((_NW, _NP), jnp.float32))
        scratch.append(pltpu.VMEM((_NP,), jnp.float32))
    scratch.append(pltpu.VMEM_SHARED((_NP, _D), jnp.float32))
    scratch.append(pltpu.SemaphoreType.DMA)
    scratch.append(pltpu.SemaphoreType.DMA)
    scratch.append(pltpu.SemaphoreType.DMA)
    scratch.append(pltpu.SemaphoreType.DMA)

    k = pl.kernel(
        functools.partial(_segsum_body, with_deg),
        out_type=tuple(out_type),
        mesh=mesh,
        scratch_types=tuple(scratch),
        compiler_params=pltpu.CompilerParams(needs_layout_passes=False),
    )
    return k(x, src, dst)


def _layer1_body(x_ref, s_ref, d_ref, wl_ref, wr_ref, b_ref, o_ref):
    s = s_ref[0] + s_ref[1]
    deg = jnp.sum(d_ref[...], axis=1, keepdims=True)
    agg = s / jnp.maximum(deg, 1.0)
    out = (jnp.dot(x_ref[...], wl_ref[...], preferred_element_type=jnp.float32)
           + jnp.dot(agg, wr_ref[...], preferred_element_type=jnp.float32)
           + b_ref[...])
    norm = jnp.sqrt(jnp.sum(out * out, axis=1, keepdims=True))
    out = out / jnp.maximum(norm, 1e-12)
    o_ref[...] = jnp.maximum(out, 0.0)


def _layer2_body(h_ref, s_ref, d_ref, wl_ref, wr_ref, b_ref,
                 wp1_ref, bp1_ref, wp2_ref, bp2_ref, o_ref):
    s = s_ref[0] + s_ref[1]
    deg = jnp.sum(d_ref[...], axis=1, keepdims=True)
    agg = s / jnp.maximum(deg, 1.0)
    out = (jnp.dot(h_ref[...], wl_ref[...], preferred_element_type=jnp.float32)
           + jnp.dot(agg, wr_ref[...], preferred_element_type=jnp.float32)
           + b_ref[...])
    norm = jnp.sqrt(jnp.sum(out * out, axis=1, keepdims=True))
    out = out / jnp.maximum(norm, 1e-12)
    out = jnp.maximum(out, 0.0)
    m = jnp.dot(out, wp1_ref[...], preferred_element_type=jnp.float32) + bp1_ref[...]
    z = jnp.dot(m, wp2_ref[...], preferred_element_type=jnp.float32) + bp2_ref[...]
    zm = z - jnp.max(z, axis=1, keepdims=True)
    o_ref[...] = zm - jnp.log(jnp.sum(jnp.exp(zm), axis=1, keepdims=True))


_BLK = 1000
_GRID = _N // _BLK


def _layer1(x, s1, degp, Wl, Wr, b):
    return pl.pallas_call(
        _layer1_body,
        grid=(_GRID,),
        in_specs=[
            pl.BlockSpec((_BLK, _D), lambda i: (i, 0)),
            pl.BlockSpec((_NC, _BLK, _D), lambda i: (0, i, 0)),
            pl.BlockSpec((_BLK, _NW), lambda i: (i, 0)),
            pl.BlockSpec((_D, _D), lambda i: (0, 0)),
            pl.BlockSpec((_D, _D), lambda i: (0, 0)),
            pl.BlockSpec((1, _D), lambda i: (0, 0)),
        ],
        out_specs=pl.BlockSpec((_BLK, _D), lambda i: (i, 0)),
        out_shape=jax.ShapeDtypeStruct((_N, _D), jnp.float32),
    )(x, s1, degp, Wl, Wr, b)


def _layer2(h, s2, degp, Wl, Wr, b, Wp1, bp1, Wp2, bp2):
    return pl.pallas_call(
        _layer2_body,
        grid=(_GRID,),
        in_specs=[
            pl.BlockSpec((_BLK, _D), lambda i: (i, 0)),
            pl.BlockSpec((_NC, _BLK, _D), lambda i: (0, i, 0)),
            pl.BlockSpec((_BLK, _NW), lambda i: (i, 0)),
            pl.BlockSpec((_D, _D), lambda i: (0, 0)),
            pl.BlockSpec((_D, _D), lambda i: (0, 0)),
            pl.BlockSpec((1, _D), lambda i: (0, 0)),
            pl.BlockSpec((_D, _D), lambda i: (0, 0)),
            pl.BlockSpec((1, _D), lambda i: (0, 0)),
            pl.BlockSpec((_D, _DOUT), lambda i: (0, 0)),
            pl.BlockSpec((1, _DOUT), lambda i: (0, 0)),
        ],
        out_specs=pl.BlockSpec((_BLK, _DOUT), lambda i: (i, 0)),
        out_shape=jax.ShapeDtypeStruct((_N, _DOUT), jnp.float32),
    )(h, s2, degp, Wl, Wr, b, Wp1, bp1, Wp2, bp2)


def kernel(x, edge_index, batch, Wl1, bl1, Wr1, br1, Wl2, bl2, Wr2, br2,
           Wp1, bp1, Wp2, bp2):
    src = edge_index[0]
    dst = edge_index[1]
    s1, degp = _segsum(x, src, dst, True)
    degp = degp.T
    h = _layer1(x, s1, degp, Wl1, Wr1, (bl1 + br1).reshape(1, _D))
    (s2,) = _segsum(h, src, dst, False)
    return _layer2(h, s2, degp, Wl2, Wr2, (bl2 + br2).reshape(1, _D),
                   Wp1, bp1.reshape(1, _D), Wp2, bp2.reshape(1, _DOUT))


# speedup vs baseline: 9.7743x; 1.0215x over previous
import functools

import jax
import jax.numpy as jnp
from jax import lax
from jax.experimental import pallas as pl
from jax.experimental.pallas import tpu as pltpu
from jax.experimental.pallas import tpu_sc as plsc

_N = 10000
_E = 320000
_D = 128
_DOUT = 64

_NC = 2
_NS = 16
_NW = _NC * _NS
_EW = _E // _NW
_CHUNK = 80
_CPS = 25
_NSEG = _EW // (_CPS * _CHUNK)
_NP = 10240
_RPT = _NP // _NS
_DEGW = 16

def _segsum_body(with_deg, *refs):
    if with_deg:
        (x_hbm, src_hbm, dst_hbm, out_hbm, deg_hbm,
         srcf_v, dstf_v, dst_v, row2_v, deg_v, acc_sh,
         gsem0, gsem1, ssem0, ssem1, stsem) = refs
    else:
        (x_hbm, src_hbm, dst_hbm, out_hbm,
         srcf_v, dstf_v, dst_v, row2_v, acc_sh,
         gsem0, gsem1, ssem0, ssem1, stsem) = refs
    row_v = row2_v.at[0]

    cid = lax.axis_index("c")
    sid = lax.axis_index("s")
    wid = sid * _NC + cid
    base = wid * _EW

    def zb_body(i, _):
        r = i // 8
        c = (i % 8) * 16
        row_v[r, pl.ds(c, 16)] = jnp.zeros((16,), jnp.float32)
        return 0
    lax.fori_loop(0, _CHUNK * 8, zb_body, 0)

    if with_deg:
        def zd_body(i, _):
            deg_v[pl.ds(i * 16, 16)] = jnp.zeros((16,), jnp.float32)
            return 0
        lax.fori_loop(0, _NP // 16, zd_body, 0)

    for k in range(_RPT // _CHUNK):
        pltpu.sync_copy(row_v, acc_sh.at[pl.ds(sid * _RPT + k * _CHUNK, _CHUNK)])
    plsc.subcore_barrier()

    _SEGE = _CPS * _CHUNK
    ones16 = jnp.ones((16,), jnp.float32)
    gsems = (gsem0, gsem1)
    ssems = (ssem0, ssem1)

    def mv_build(p):
        def mv_body(i, _):
            r = i // (_CHUNK // 16)
            c = (i % (_CHUNK // 16)) * 16
            dst_v[p * _CPS + r, pl.ds(c, 16)] = dstf_v[pl.ds(p * _SEGE + i * 16, 16)]
            return 0
        lax.fori_loop(0, _SEGE // 16, mv_body, 0)

    pltpu.sync_copy(src_hbm.at[pl.ds(base, _SEGE)], srcf_v.at[pl.ds(0, _SEGE)])
    pltpu.sync_copy(dst_hbm.at[pl.ds(base, _SEGE)], dstf_v.at[pl.ds(0, _SEGE)])
    mv_build(0)

    def seg_body(s, _):
        par = lax.rem(s, 2)
        nxt = 1 - par
        sn = jnp.minimum(s + 1, _NSEG - 1)
        nxt_base = base + sn * _SEGE
        pltpu.async_copy(src_hbm.at[pl.ds(nxt_base, _SEGE)],
                         srcf_v.at[pl.ds(nxt * _SEGE, _SEGE)], stsem)
        pltpu.async_copy(dst_hbm.at[pl.ds(nxt_base, _SEGE)],
                         dstf_v.at[pl.ds(nxt * _SEGE, _SEGE)], stsem)

        def gissue(j, b):
            jc = jnp.minimum(j, _CPS - 1)
            pltpu.async_copy(
                x_hbm.at[srcf_v.at[pl.ds(par * _SEGE + jc * _CHUNK, _CHUNK)]],
                row2_v.at[b], gsems[b])

        def gwait(j, b):
            jc = jnp.minimum(j, _CPS - 1)
            pltpu.make_async_copy(
                x_hbm.at[srcf_v.at[pl.ds(par * _SEGE + jc * _CHUNK, _CHUNK)]],
                row2_v.at[b], gsems[b]).wait()

        def sissue(j, b):
            pltpu.async_copy(row2_v.at[b], acc_sh.at[dst_v.at[par * _CPS + j]],
                             ssems[b], add=True)
            if with_deg:
                for g in range(_CHUNK // 16):
                    idx = dst_v[par * _CPS + j, pl.ds(g * 16, 16)]
                    plsc.addupdate_scatter(deg_v, [idx], ones16)

        def swait(j, b):
            pltpu.make_async_copy(row2_v.at[b], acc_sh.at[dst_v.at[par * _CPS + j]],
                                  ssems[b]).wait()

        gissue(0, 0)
        gissue(1, 1)

        def pipe_body(jj, _):
            j0 = jj * 2
            gwait(j0, 0)
            sissue(j0, 0)
            gwait(j0 + 1, 1)
            sissue(j0 + 1, 1)
            swait(j0, 0)
            gissue(j0 + 2, 0)
            swait(j0 + 1, 1)
            gissue(j0 + 3, 1)
            return 0
        lax.fori_loop(0, (_CPS - 1) // 2, pipe_body, 0)

        gwait(_CPS - 1, 0)
        sissue(_CPS - 1, 0)
        gwait(_CPS - 1, 1)
        swait(_CPS - 1, 0)

        pltpu.make_async_copy(src_hbm.at[pl.ds(nxt_base, _SEGE)],
                              srcf_v.at[pl.ds(nxt * _SEGE, _SEGE)], stsem).wait()
        pltpu.make_async_copy(dst_hbm.at[pl.ds(nxt_base, _SEGE)],
                              dstf_v.at[pl.ds(nxt * _SEGE, _SEGE)], stsem).wait()
        mv_build(nxt)
        return 0
    lax.fori_loop(0, _NSEG, seg_body, 0)

    plsc.subcore_barrier()

    pltpu.sync_copy(acc_sh.at[pl.ds(sid * _RPT, _RPT)],
                    out_hbm.at[cid, pl.ds(sid * _RPT, _RPT)])
    if with_deg:
        pltpu.sync_copy(deg_v, deg_hbm.at[wid])


def _segsum(x, src, dst, with_deg):
    mesh = plsc.VectorSubcoreMesh(core_axis_name="c", subcore_axis_name="s")
    out_type = [jax.ShapeDtypeStruct((_NC, _NP, _D), jnp.float32)]
    scratch = [
        pltpu.VMEM((2 * _CPS * _CHUNK,), jnp.int32),
        pltpu.VMEM((2 * _CPS * _CHUNK,), jnp.int32),
        pltpu.VMEM((2 * _CPS, _CHUNK), jnp.int32),
        pltpu.VMEM((2, _CHUNK, _D), jnp.float32),
    ]
    if with_deg:
        out_type.append(jax.ShapeDtypeStruct((_NW, _NP), jnp.float32))
        scratch.append(pltpu.VMEM((_NP,), jnp.float32))
    scratch.append(pltpu.VMEM_SHARED((_NP, _D), jnp.float32))
    scratch.append(pltpu.SemaphoreType.DMA)
    scratch.append(pltpu.SemaphoreType.DMA)
    scratch.append(pltpu.SemaphoreType.DMA)
    scratch.append(pltpu.SemaphoreType.DMA)
    scratch.append(pltpu.SemaphoreType.DMA)

    k = pl.kernel(
        functools.partial(_segsum_body, with_deg),
        out_type=tuple(out_type),
        mesh=mesh,
        scratch_types=tuple(scratch),
        compiler_params=pltpu.CompilerParams(needs_layout_passes=False),
    )
    return k(x, src, dst)


def _layer1_body(x_ref, s_ref, d_ref, wl_ref, wr_ref, b_ref, o_ref):
    s = s_ref[0] + s_ref[1]
    deg = jnp.sum(d_ref[...], axis=1, keepdims=True)
    agg = s / jnp.maximum(deg, 1.0)
    out = (jnp.dot(x_ref[...], wl_ref[...], preferred_element_type=jnp.float32)
           + jnp.dot(agg, wr_ref[...], preferred_element_type=jnp.float32)
           + b_ref[...])
    norm = jnp.sqrt(jnp.sum(out * out, axis=1, keepdims=True))
    out = out / jnp.maximum(norm, 1e-12)
    o_ref[...] = jnp.maximum(out, 0.0)


def _layer2_body(h_ref, s_ref, d_ref, wl_ref, wr_ref, b_ref,
                 wp1_ref, bp1_ref, wp2_ref, bp2_ref, o_ref):
    s = s_ref[0] + s_ref[1]
    deg = jnp.sum(d_ref[...], axis=1, keepdims=True)
    agg = s / jnp.maximum(deg, 1.0)
    out = (jnp.dot(h_ref[...], wl_ref[...], preferred_element_type=jnp.float32)
           + jnp.dot(agg, wr_ref[...], preferred_element_type=jnp.float32)
           + b_ref[...])
    norm = jnp.sqrt(jnp.sum(out * out, axis=1, keepdims=True))
    out = out / jnp.maximum(norm, 1e-12)
    out = jnp.maximum(out, 0.0)
    m = jnp.dot(out, wp1_ref[...], preferred_element_type=jnp.float32) + bp1_ref[...]
    z = jnp.dot(m, wp2_ref[...], preferred_element_type=jnp.float32) + bp2_ref[...]
    zm = z - jnp.max(z, axis=1, keepdims=True)
    o_ref[...] = zm - jnp.log(jnp.sum(jnp.exp(zm), axis=1, keepdims=True))


_BLK = 1000
_GRID = _N // _BLK


def _layer1(x, s1, degp, Wl, Wr, b):
    return pl.pallas_call(
        _layer1_body,
        grid=(_GRID,),
        in_specs=[
            pl.BlockSpec((_BLK, _D), lambda i: (i, 0)),
            pl.BlockSpec((_NC, _BLK, _D), lambda i: (0, i, 0)),
            pl.BlockSpec((_BLK, _NW), lambda i: (i, 0)),
            pl.BlockSpec((_D, _D), lambda i: (0, 0)),
            pl.BlockSpec((_D, _D), lambda i: (0, 0)),
            pl.BlockSpec((1, _D), lambda i: (0, 0)),
        ],
        out_specs=pl.BlockSpec((_BLK, _D), lambda i: (i, 0)),
        out_shape=jax.ShapeDtypeStruct((_N, _D), jnp.float32),
    )(x, s1, degp, Wl, Wr, b)


def _layer2(h, s2, degp, Wl, Wr, b, Wp1, bp1, Wp2, bp2):
    return pl.pallas_call(
        _layer2_body,
        grid=(_GRID,),
        in_specs=[
            pl.BlockSpec((_BLK, _D), lambda i: (i, 0)),
            pl.BlockSpec((_NC, _BLK, _D), lambda i: (0, i, 0)),
            pl.BlockSpec((_BLK, _NW), lambda i: (i, 0)),
            pl.BlockSpec((_D, _D), lambda i: (0, 0)),
            pl.BlockSpec((_D, _D), lambda i: (0, 0)),
            pl.BlockSpec((1, _D), lambda i: (0, 0)),
            pl.BlockSpec((_D, _D), lambda i: (0, 0)),
            pl.BlockSpec((1, _D), lambda i: (0, 0)),
            pl.BlockSpec((_D, _DOUT), lambda i: (0, 0)),
            pl.BlockSpec((1, _DOUT), lambda i: (0, 0)),
        ],
        out_specs=pl.BlockSpec((_BLK, _DOUT), lambda i: (i, 0)),
        out_shape=jax.ShapeDtypeStruct((_N, _DOUT), jnp.float32),
    )(h, s2, degp, Wl, Wr, b, Wp1, bp1, Wp2, bp2)


def kernel(x, edge_index, batch, Wl1, bl1, Wr1, br1, Wl2, bl2, Wr2, br2,
           Wp1, bp1, Wp2, bp2):
    src = edge_index[0]
    dst = edge_index[1]
    s1, degp = _segsum(x, src, dst, True)
    degp = degp.T
    h = _layer1(x, s1, degp, Wl1, Wr1, (bl1 + br1).reshape(1, _D))
    (s2,) = _segsum(h, src, dst, False)
    return _layer2(h, s2, degp, Wl2, Wr2, (bl2 + br2).reshape(1, _D),
                   Wp1, bp1.reshape(1, _D), Wp2, bp2.reshape(1, _DOUT))
